# baseline (speedup 1.0000x reference)
import functools

import jax
import jax.numpy as jnp
from jax import lax
from jax.experimental import pallas as pl
from jax.experimental.pallas import tpu as pltpu
from jax.experimental.pallas import tpu_sc as plsc

N = 10000
E = 320000
D = 128

NC = 2
NS = 16
NW = NC * NS
NWH = NW // 2
EPW = E // NWH
CH = 80
NCH = EPW // CH
IB = 10
NB = NCH // IB
N_PAD = 10112
RPT = N_PAD // NS


def _sc_body(x_hbm, us_hbm, ud_hbm, ua_hbm, ds_hbm, dd_hbm, da_hbm, z_hbm,
             out0, out1,
             acc, idx_src, idx_dst, xb0, ab0, xb1, ab1, sem_g, sem_a, sem_s):
    c = lax.axis_index("c")
    s = lax.axis_index("s")
    w = s * NC + c
    t = s

    pltpu.sync_copy(z_hbm, acc.at[pl.ds(t * RPT, RPT)])
    plsc.subcore_barrier()

    def run_pipeline(src_hbm, dst_hbm, attr_hbm, wl):
        def start(b, j, xb, ab):
            pltpu.async_copy(x_hbm.at[idx_src.at[j]], xb, sem_g)
            pltpu.async_copy(
                attr_hbm.at[wl, pl.ds((b * IB + j) * CH, CH)], ab, sem_a)

        def wait_gathers(b, j, xb, ab):
            pltpu.make_async_copy(x_hbm.at[idx_src.at[j]], xb, sem_g).wait()
            pltpu.make_async_copy(
                attr_hbm.at[wl, pl.ds((b * IB + j) * CH, CH)], ab,
                sem_a).wait()

        def scatter(j, xb, ab):
            pltpu.async_copy(xb, acc.at[idx_dst.at[j]], sem_s, add=True)
            pltpu.async_copy(ab, acc.at[idx_dst.at[j]], sem_s, add=True)

        def drain_scatters(j, xb, ab):
            pltpu.make_async_copy(xb, acc.at[idx_dst.at[j]], sem_s).wait()
            pltpu.make_async_copy(ab, acc.at[idx_dst.at[j]], sem_s).wait()

        def pair_body(b, i, carry):
            @pl.when(i > 0)
            def _():
                drain_scatters(2 * i - 1, xb1, ab1)
            start(b, 2 * i + 1, xb1, ab1)
            wait_gathers(b, 2 * i, xb0, ab0)
            scatter(2 * i, xb0, ab0)

            @pl.when(i < IB // 2 - 1)
            def _():
                drain_scatters(2 * i, xb0, ab0)
                start(b, 2 * i + 2, xb0, ab0)
            wait_gathers(b, 2 * i + 1, xb1, ab1)
            scatter(2 * i + 1, xb1, ab1)
            return carry

        def block_body(b, carry):
            pltpu.sync_copy(src_hbm.at[wl, b], idx_src)
            pltpu.sync_copy(dst_hbm.at[wl, b], idx_dst)
            start(b, 0, xb0, ab0)
            r = lax.fori_loop(0, IB // 2,
                              lambda i, cr: pair_body(b, i, cr), carry)
            drain_scatters(IB - 2, xb0, ab0)
            drain_scatters(IB - 1, xb1, ab1)
            return r

        lax.fori_loop(0, NB, block_body, 0)

    @pl.when(w < NWH)
    def _():
        run_pipeline(us_hbm, ud_hbm, ua_hbm, w)

    @pl.when(w >= NWH)
    def _():
        run_pipeline(ds_hbm, dd_hbm, da_hbm, w - NWH)

    plsc.subcore_barrier()

    @pl.when(c == 0)
    def _():
        pltpu.sync_copy(acc.at[pl.ds(t * RPT, RPT)],
                        out0.at[pl.ds(t * RPT, RPT)])

    @pl.when(c == 1)
    def _():
        pltpu.sync_copy(acc.at[pl.ds(t * RPT, RPT)],
                        out1.at[pl.ds(t * RPT, RPT)])


def _combine_body(x_ref, a_ref, b_ref, o_ref):
    o_ref[...] = x_ref[...] + a_ref[...] + b_ref[...]


def kernel(x, up_index, up_attr, down_index, down_attr):
    us = up_index[0].astype(jnp.int32).reshape(NWH, NB, IB, CH)
    ud = up_index[1].astype(jnp.int32).reshape(NWH, NB, IB, CH)
    ds_ = down_index[0].astype(jnp.int32).reshape(NWH, NB, IB, CH)
    dd = down_index[1].astype(jnp.int32).reshape(NWH, NB, IB, CH)
    ua = up_attr.reshape(NWH, EPW, D)
    da = down_attr.reshape(NWH, EPW, D)
    zeros = jnp.zeros((RPT, D), jnp.float32)

    mesh = plsc.VectorSubcoreMesh(core_axis_name="c", subcore_axis_name="s")
    scatter = pl.kernel(
        _sc_body,
        mesh=mesh,
        out_type=[jax.ShapeDtypeStruct((N_PAD, D), jnp.float32),
                  jax.ShapeDtypeStruct((N_PAD, D), jnp.float32)],
        scratch_types=[
            pltpu.VMEM_SHARED((N_PAD, D), jnp.float32),
            pltpu.VMEM((IB, CH), jnp.int32),
            pltpu.VMEM((IB, CH), jnp.int32),
            pltpu.VMEM((CH, D), jnp.float32),
            pltpu.VMEM((CH, D), jnp.float32),
            pltpu.VMEM((CH, D), jnp.float32),
            pltpu.VMEM((CH, D), jnp.float32),
            pltpu.SemaphoreType.DMA,
            pltpu.SemaphoreType.DMA,
            pltpu.SemaphoreType.DMA,
        ],
    )
    a0, a1 = scatter(x, us, ud, ua, ds_, dd, da, zeros)

    out = pl.pallas_call(
        _combine_body,
        grid=(1,),
        in_specs=[pl.BlockSpec((N, D), lambda i: (0, 0))] * 3,
        out_specs=pl.BlockSpec((N, D), lambda i: (0, 0)),
        out_shape=jax.ShapeDtypeStruct((N, D), jnp.float32),
    )(x, a0, a1)
    return out


# speedup vs baseline: 1.0580x; 1.0580x over previous
import functools

import jax
import jax.numpy as jnp
from jax import lax
from jax.experimental import pallas as pl
from jax.experimental.pallas import tpu as pltpu
from jax.experimental.pallas import tpu_sc as plsc

N = 10000
E = 320000
D = 128

NC = 2
NS = 16
NW = NC * NS
NWH = NW // 2
EPW = E // NWH
CH = 80
NCH = EPW // CH
IB = 10
NB = NCH // IB
N_PAD = 10112
RPT = N_PAD // NS


def _sc_body(x_hbm, us_hbm, ud_hbm, ua_hbm, ds_hbm, dd_hbm, da_hbm, z_hbm,
             out0, out1,
             acc, ia_src, ia_dst, ib_src, ib_dst, xb0, ab0, xb1, ab1,
             sem_g, sem_a, sem_s, sem_i):
    c = lax.axis_index("c")
    s = lax.axis_index("s")
    w = s * NC + c
    t = s

    pltpu.sync_copy(z_hbm, acc.at[pl.ds(t * RPT, RPT)])
    plsc.subcore_barrier()

    def run_pipeline(src_hbm, dst_hbm, attr_hbm, wl):
        def start(b, j, i_src, xb, ab):
            pltpu.async_copy(x_hbm.at[i_src.at[j]], xb, sem_g)
            pltpu.async_copy(
                attr_hbm.at[wl, pl.ds((b * IB + j) * CH, CH)], ab, sem_a)

        def wait_gathers(b, j, i_src, xb, ab):
            pltpu.make_async_copy(x_hbm.at[i_src.at[j]], xb, sem_g).wait()
            pltpu.make_async_copy(
                attr_hbm.at[wl, pl.ds((b * IB + j) * CH, CH)], ab,
                sem_a).wait()

        def scatter(j, i_dst, xb, ab):
            pltpu.async_copy(xb, acc.at[i_dst.at[j]], sem_s, add=True)
            pltpu.async_copy(ab, acc.at[i_dst.at[j]], sem_s, add=True)

        def drain_scatters(j, i_dst, xb, ab):
            pltpu.make_async_copy(xb, acc.at[i_dst.at[j]], sem_s).wait()
            pltpu.make_async_copy(ab, acc.at[i_dst.at[j]], sem_s).wait()

        def prefetch_idx(b, i_src, i_dst):
            pltpu.async_copy(src_hbm.at[wl, b], i_src, sem_i)
            pltpu.async_copy(dst_hbm.at[wl, b], i_dst, sem_i)

        def wait_idx(b, i_src, i_dst):
            pltpu.make_async_copy(src_hbm.at[wl, b], i_src, sem_i).wait()
            pltpu.make_async_copy(dst_hbm.at[wl, b], i_dst, sem_i).wait()

        def process_block(b, i_src, i_dst, carry):
            def pair_body(i, cr):
                @pl.when(i > 0)
                def _():
                    drain_scatters(2 * i - 1, i_dst, xb1, ab1)
                start(b, 2 * i + 1, i_src, xb1, ab1)
                wait_gathers(b, 2 * i, i_src, xb0, ab0)
                scatter(2 * i, i_dst, xb0, ab0)

                @pl.when(i < IB // 2 - 1)
                def _():
                    drain_scatters(2 * i, i_dst, xb0, ab0)
                    start(b, 2 * i + 2, i_src, xb0, ab0)
                wait_gathers(b, 2 * i + 1, i_src, xb1, ab1)
                scatter(2 * i + 1, i_dst, xb1, ab1)
                return cr

            start(b, 0, i_src, xb0, ab0)
            r = lax.fori_loop(0, IB // 2, pair_body, carry)
            drain_scatters(IB - 2, i_dst, xb0, ab0)
            drain_scatters(IB - 1, i_dst, xb1, ab1)
            return r

        pltpu.sync_copy(src_hbm.at[wl, 0], ia_src)
        pltpu.sync_copy(dst_hbm.at[wl, 0], ia_dst)
        prefetch_idx(1, ib_src, ib_dst)

        def block_pair(p, carry):
            b = 2 * p
            r = process_block(b, ia_src, ia_dst, carry)
            wait_idx(b + 1, ib_src, ib_dst)
            prefetch_idx(b + 2, ia_src, ia_dst)
            r = process_block(b + 1, ib_src, ib_dst, r)
            wait_idx(b + 2, ia_src, ia_dst)

            @pl.when(p < NB // 2 - 1)
            def _():
                prefetch_idx(b + 3, ib_src, ib_dst)
            return r

        r = lax.fori_loop(0, NB // 2, block_pair, 0)
        process_block(NB - 1, ia_src, ia_dst, r)

    @pl.when(w < NWH)
    def _():
        run_pipeline(us_hbm, ud_hbm, ua_hbm, w)

    @pl.when(w >= NWH)
    def _():
        run_pipeline(ds_hbm, dd_hbm, da_hbm, w - NWH)

    plsc.subcore_barrier()

    @pl.when(c == 0)
    def _():
        pltpu.sync_copy(acc.at[pl.ds(t * RPT, RPT)],
                        out0.at[pl.ds(t * RPT, RPT)])

    @pl.when(c == 1)
    def _():
        pltpu.sync_copy(acc.at[pl.ds(t * RPT, RPT)],
                        out1.at[pl.ds(t * RPT, RPT)])


def _combine_body(x_ref, a_ref, b_ref, o_ref):
    o_ref[...] = x_ref[...] + a_ref[...] + b_ref[...]


def kernel(x, up_index, up_attr, down_index, down_attr):
    us = up_index[0].astype(jnp.int32).reshape(NWH, NB, IB, CH)
    ud = up_index[1].astype(jnp.int32).reshape(NWH, NB, IB, CH)
    ds_ = down_index[0].astype(jnp.int32).reshape(NWH, NB, IB, CH)
    dd = down_index[1].astype(jnp.int32).reshape(NWH, NB, IB, CH)
    ua = up_attr.reshape(NWH, EPW, D)
    da = down_attr.reshape(NWH, EPW, D)
    zeros = jnp.zeros((RPT, D), jnp.float32)

    mesh = plsc.VectorSubcoreMesh(core_axis_name="c", subcore_axis_name="s")
    scatter = pl.kernel(
        _sc_body,
        mesh=mesh,
        out_type=[jax.ShapeDtypeStruct((N_PAD, D), jnp.float32),
                  jax.ShapeDtypeStruct((N_PAD, D), jnp.float32)],
        scratch_types=[
            pltpu.VMEM_SHARED((N_PAD, D), jnp.float32),
            pltpu.VMEM((IB, CH), jnp.int32),
            pltpu.VMEM((IB, CH), jnp.int32),
            pltpu.VMEM((IB, CH), jnp.int32),
            pltpu.VMEM((IB, CH), jnp.int32),
            pltpu.VMEM((CH, D), jnp.float32),
            pltpu.VMEM((CH, D), jnp.float32),
            pltpu.VMEM((CH, D), jnp.float32),
            pltpu.VMEM((CH, D), jnp.float32),
            pltpu.SemaphoreType.DMA,
            pltpu.SemaphoreType.DMA,
            pltpu.SemaphoreType.DMA,
            pltpu.SemaphoreType.DMA,
        ],
    )
    a0, a1 = scatter(x, us, ud, ua, ds_, dd, da, zeros)

    out = pl.pallas_call(
        _combine_body,
        grid=(1,),
        in_specs=[pl.BlockSpec((N, D), lambda i: (0, 0))] * 3,
        out_specs=pl.BlockSpec((N, D), lambda i: (0, 0)),
        out_shape=jax.ShapeDtypeStruct((N, D), jnp.float32),
    )(x, a0, a1)
    return out


# speedup vs baseline: 1.0991x; 1.0388x over previous
import functools

import jax
import jax.numpy as jnp
from jax import lax
from jax.experimental import pallas as pl
from jax.experimental.pallas import tpu as pltpu
from jax.experimental.pallas import tpu_sc as plsc

N = 10000
E = 320000
D = 128

NC = 2
NS = 16
NW = NC * NS
NWH = NW // 2
EPW = E // NWH
CH = 80
NCH = EPW // CH
IB = 10
NB = NCH // IB
N_PAD = 10112
RPT = N_PAD // NS


def _sc_body(x_hbm, us_hbm, ud_hbm, ua_hbm, ds_hbm, dd_hbm, da_hbm, z_hbm,
             out0, out1,
             acc, ia_src, ia_dst, ib_src, ib_dst, xb0, ab0, xb1, ab1,
             sem_g, sem_a, sem_s, sem_i):
    c = lax.axis_index("c")
    s = lax.axis_index("s")
    w = s * NC + c
    t = s

    pltpu.sync_copy(z_hbm, acc.at[pl.ds(t * RPT, RPT)])
    plsc.subcore_barrier()

    def run_pipeline(src_hbm, dst_hbm, attr_hbm, wl):
        def start(b, j, i_src, xb, ab):
            pltpu.async_copy(x_hbm.at[i_src.at[j]], xb, sem_g)
            pltpu.async_copy(
                attr_hbm.at[wl, pl.ds((b * IB + j) * CH, CH)], ab, sem_a)

        def wait_gathers(b, j, i_src, xb, ab):
            pltpu.make_async_copy(x_hbm.at[i_src.at[j]], xb, sem_g).wait()
            pltpu.make_async_copy(
                attr_hbm.at[wl, pl.ds((b * IB + j) * CH, CH)], ab,
                sem_a).wait()

        def scatter(j, i_dst, xb, ab):
            pltpu.async_copy(xb, acc.at[i_dst.at[j]], sem_s, add=True)
            pltpu.async_copy(ab, acc.at[i_dst.at[j]], sem_s, add=True)

        def drain_scatters(j, i_dst, xb, ab):
            pltpu.make_async_copy(xb, acc.at[i_dst.at[j]], sem_s).wait()
            pltpu.make_async_copy(ab, acc.at[i_dst.at[j]], sem_s).wait()

        def prefetch_idx(b, i_src, i_dst):
            pltpu.async_copy(src_hbm.at[wl, b], i_src, sem_i)
            pltpu.async_copy(dst_hbm.at[wl, b], i_dst, sem_i)

        def wait_idx(b, i_src, i_dst):
            pltpu.make_async_copy(src_hbm.at[wl, b], i_src, sem_i).wait()
            pltpu.make_async_copy(dst_hbm.at[wl, b], i_dst, sem_i).wait()

        def process_block(b, i_src, i_dst, primed):
            def pair_body(i, cr):
                @pl.when(i > 0)
                def _():
                    drain_scatters(2 * i - 1, i_dst, xb1, ab1)
                start(b, 2 * i + 1, i_src, xb1, ab1)
                wait_gathers(b, 2 * i, i_src, xb0, ab0)
                scatter(2 * i, i_dst, xb0, ab0)

                @pl.when(i < IB // 2 - 1)
                def _():
                    drain_scatters(2 * i, i_dst, xb0, ab0)
                    start(b, 2 * i + 2, i_src, xb0, ab0)
                wait_gathers(b, 2 * i + 1, i_src, xb1, ab1)
                scatter(2 * i + 1, i_dst, xb1, ab1)
                return cr

            if not primed:
                start(b, 0, i_src, xb0, ab0)
            lax.fori_loop(0, IB // 2, pair_body, 0)

        def finish_block(b, i_dst, nxt_b, nxt_src, prime):
            drain_scatters(IB - 2, i_dst, xb0, ab0)
            if prime:
                start(nxt_b, 0, nxt_src, xb0, ab0)
            drain_scatters(IB - 1, i_dst, xb1, ab1)

        pltpu.sync_copy(src_hbm.at[wl, 0], ia_src)
        pltpu.sync_copy(dst_hbm.at[wl, 0], ia_dst)
        prefetch_idx(1, ib_src, ib_dst)
        process_block(0, ia_src, ia_dst, False)
        wait_idx(1, ib_src, ib_dst)
        finish_block(0, ia_dst, 1, ib_src, True)
        prefetch_idx(2, ia_src, ia_dst)

        def block_pair(p, carry):
            b1 = 2 * p + 1
            process_block(b1, ib_src, ib_dst, True)
            wait_idx(b1 + 1, ia_src, ia_dst)
            finish_block(b1, ib_dst, b1 + 1, ia_src, True)

            @pl.when(p < NB // 2 - 1)
            def _():
                prefetch_idx(b1 + 2, ib_src, ib_dst)
            b2 = b1 + 1
            process_block(b2, ia_src, ia_dst, True)

            @pl.when(p < NB // 2 - 1)
            def _():
                wait_idx(b2 + 1, ib_src, ib_dst)
                finish_block(b2, ia_dst, b2 + 1, ib_src, True)
                prefetch_idx(b2 + 2, ia_src, ia_dst)

            @pl.when(p == NB // 2 - 1)
            def _():
                finish_block(b2, ia_dst, 0, ia_src, False)
            return carry

        lax.fori_loop(0, NB // 2, block_pair, 0)

    @pl.when(w < NWH)
    def _():
        run_pipeline(us_hbm, ud_hbm, ua_hbm, w)

    @pl.when(w >= NWH)
    def _():
        run_pipeline(ds_hbm, dd_hbm, da_hbm, w - NWH)

    plsc.subcore_barrier()

    @pl.when(c == 0)
    def _():
        pltpu.sync_copy(acc.at[pl.ds(t * RPT, RPT)],
                        out0.at[pl.ds(t * RPT, RPT)])

    @pl.when(c == 1)
    def _():
        pltpu.sync_copy(acc.at[pl.ds(t * RPT, RPT)],
                        out1.at[pl.ds(t * RPT, RPT)])


def _combine_body(x_ref, a_ref, b_ref, o_ref):
    o_ref[...] = x_ref[...] + a_ref[...] + b_ref[...]


def kernel(x, up_index, up_attr, down_index, down_attr):
    us = up_index[0].astype(jnp.int32).reshape(NWH, NB, IB, CH)
    ud = up_index[1].astype(jnp.int32).reshape(NWH, NB, IB, CH)
    ds_ = down_index[0].astype(jnp.int32).reshape(NWH, NB, IB, CH)
    dd = down_index[1].astype(jnp.int32).reshape(NWH, NB, IB, CH)
    ua = up_attr.reshape(NWH, EPW, D)
    da = down_attr.reshape(NWH, EPW, D)
    zeros = jnp.zeros((RPT, D), jnp.float32)

    mesh = plsc.VectorSubcoreMesh(core_axis_name="c", subcore_axis_name="s")
    scatter = pl.kernel(
        _sc_body,
        mesh=mesh,
        out_type=[jax.ShapeDtypeStruct((N_PAD, D), jnp.float32),
                  jax.ShapeDtypeStruct((N_PAD, D), jnp.float32)],
        scratch_types=[
            pltpu.VMEM_SHARED((N_PAD, D), jnp.float32),
            pltpu.VMEM((IB, CH), jnp.int32),
            pltpu.VMEM((IB, CH), jnp.int32),
            pltpu.VMEM((IB, CH), jnp.int32),
            pltpu.VMEM((IB, CH), jnp.int32),
            pltpu.VMEM((CH, D), jnp.float32),
            pltpu.VMEM((CH, D), jnp.float32),
            pltpu.VMEM((CH, D), jnp.float32),
            pltpu.VMEM((CH, D), jnp.float32),
            pltpu.SemaphoreType.DMA,
            pltpu.SemaphoreType.DMA,
            pltpu.SemaphoreType.DMA,
            pltpu.SemaphoreType.DMA,
        ],
    )
    a0, a1 = scatter(x, us, ud, ua, ds_, dd, da, zeros)

    out = pl.pallas_call(
        _combine_body,
        grid=(1,),
        in_specs=[pl.BlockSpec((N, D), lambda i: (0, 0))] * 3,
        out_specs=pl.BlockSpec((N, D), lambda i: (0, 0)),
        out_shape=jax.ShapeDtypeStruct((N, D), jnp.float32),
    )(x, a0, a1)
    return out


# speedup vs baseline: 1.1988x; 1.0908x over previous
import functools

import jax
import jax.numpy as jnp
from jax import lax
from jax.experimental import pallas as pl
from jax.experimental.pallas import tpu as pltpu
from jax.experimental.pallas import tpu_sc as plsc

N = 10000
E = 320000
D = 128

NC = 2
NS = 16
NW = NC * NS
NWH = NW // 2
EPW = E // NWH
CH = 80
NCH = EPW // CH
IB = 10
NB = NCH // IB
N_PAD = 10112
RPT = N_PAD // NS


def _sc_body(x_hbm, us_hbm, ud_hbm, ua_hbm, ds_hbm, dd_hbm, da_hbm, z_hbm,
             out0, out1,
             acc, ia_src, ia_dst, ib_src, ib_dst, xb0, ab0, xb1, ab1,
             sem_g, sem_a, sem_s, sem_i):
    c = lax.axis_index("c")
    s = lax.axis_index("s")
    w = s * NC + c
    t = s

    pltpu.sync_copy(z_hbm, acc.at[pl.ds(t * RPT, RPT)])
    plsc.subcore_barrier()

    def run_pipeline(src_hbm, dst_hbm, attr_hbm, wl):
        def start_x(b, j, i_src, xb):
            pltpu.async_copy(x_hbm.at[i_src.at[j]], xb, sem_g)

        def start_a(b, j, ab):
            pltpu.async_copy(
                attr_hbm.at[wl, pl.ds((b * IB + j) * CH, CH)], ab, sem_a)

        def start(b, j, i_src, xb, ab):
            start_x(b, j, i_src, xb)
            start_a(b, j, ab)

        def wait_gathers(b, j, i_src, xb, ab):
            pltpu.make_async_copy(x_hbm.at[i_src.at[j]], xb, sem_g).wait()
            pltpu.make_async_copy(
                attr_hbm.at[wl, pl.ds((b * IB + j) * CH, CH)], ab,
                sem_a).wait()

        def scatter(j, i_dst, xb, ab):
            pltpu.async_copy(xb, acc.at[i_dst.at[j]], sem_s, add=True)
            pltpu.async_copy(ab, acc.at[i_dst.at[j]], sem_s, add=True)

        def drain_x(j, i_dst, xb):
            pltpu.make_async_copy(xb, acc.at[i_dst.at[j]], sem_s).wait()

        def drain_a(j, i_dst, ab):
            pltpu.make_async_copy(ab, acc.at[i_dst.at[j]], sem_s).wait()

        def drain_scatters(j, i_dst, xb, ab):
            drain_x(j, i_dst, xb)
            drain_a(j, i_dst, ab)

        def prefetch_idx(b, i_src, i_dst):
            pltpu.async_copy(src_hbm.at[wl, b], i_src, sem_i)
            pltpu.async_copy(dst_hbm.at[wl, b], i_dst, sem_i)

        def wait_idx(b, i_src, i_dst):
            pltpu.make_async_copy(src_hbm.at[wl, b], i_src, sem_i).wait()
            pltpu.make_async_copy(dst_hbm.at[wl, b], i_dst, sem_i).wait()

        def process_block(b, i_src, i_dst, primed):
            def pair_body(i, cr):
                @pl.when(i > 0)
                def _():
                    drain_x(2 * i - 1, i_dst, xb1)
                start_x(b, 2 * i + 1, i_src, xb1)

                @pl.when(i > 0)
                def _():
                    drain_a(2 * i - 1, i_dst, ab1)
                start_a(b, 2 * i + 1, ab1)
                wait_gathers(b, 2 * i, i_src, xb0, ab0)
                scatter(2 * i, i_dst, xb0, ab0)

                @pl.when(i < IB // 2 - 1)
                def _():
                    drain_x(2 * i, i_dst, xb0)
                    start_x(b, 2 * i + 2, i_src, xb0)
                    drain_a(2 * i, i_dst, ab0)
                    start_a(b, 2 * i + 2, ab0)
                wait_gathers(b, 2 * i + 1, i_src, xb1, ab1)
                scatter(2 * i + 1, i_dst, xb1, ab1)
                return cr

            if not primed:
                start(b, 0, i_src, xb0, ab0)
            lax.fori_loop(0, IB // 2, pair_body, 0)

        def finish_block(b, i_dst, nxt_b, nxt_src, prime):
            drain_x(IB - 2, i_dst, xb0)
            if prime:
                start_x(nxt_b, 0, nxt_src, xb0)
            drain_a(IB - 2, i_dst, ab0)
            if prime:
                start_a(nxt_b, 0, ab0)
            drain_scatters(IB - 1, i_dst, xb1, ab1)

        pltpu.sync_copy(src_hbm.at[wl, 0], ia_src)
        pltpu.sync_copy(dst_hbm.at[wl, 0], ia_dst)
        prefetch_idx(1, ib_src, ib_dst)
        process_block(0, ia_src, ia_dst, False)
        wait_idx(1, ib_src, ib_dst)
        finish_block(0, ia_dst, 1, ib_src, True)
        prefetch_idx(2, ia_src, ia_dst)

        def block_pair(p, carry):
            b1 = 2 * p + 1
            process_block(b1, ib_src, ib_dst, True)
            wait_idx(b1 + 1, ia_src, ia_dst)
            finish_block(b1, ib_dst, b1 + 1, ia_src, True)

            @pl.when(p < NB // 2 - 1)
            def _():
                prefetch_idx(b1 + 2, ib_src, ib_dst)
            b2 = b1 + 1
            process_block(b2, ia_src, ia_dst, True)

            @pl.when(p < NB // 2 - 1)
            def _():
                wait_idx(b2 + 1, ib_src, ib_dst)
                finish_block(b2, ia_dst, b2 + 1, ib_src, True)
                prefetch_idx(b2 + 2, ia_src, ia_dst)

            @pl.when(p == NB // 2 - 1)
            def _():
                finish_block(b2, ia_dst, 0, ia_src, False)
            return carry

        lax.fori_loop(0, NB // 2, block_pair, 0)

    @pl.when(w < NWH)
    def _():
        run_pipeline(us_hbm, ud_hbm, ua_hbm, w)

    @pl.when(w >= NWH)
    def _():
        run_pipeline(ds_hbm, dd_hbm, da_hbm, w - NWH)

    plsc.subcore_barrier()

    @pl.when(c == 0)
    def _():
        pltpu.sync_copy(acc.at[pl.ds(t * RPT, RPT)],
                        out0.at[pl.ds(t * RPT, RPT)])

    @pl.when(c == 1)
    def _():
        pltpu.sync_copy(acc.at[pl.ds(t * RPT, RPT)],
                        out1.at[pl.ds(t * RPT, RPT)])


def _combine_body(x_ref, a_ref, b_ref, o_ref):
    o_ref[...] = x_ref[...] + a_ref[...] + b_ref[...]


def kernel(x, up_index, up_attr, down_index, down_attr):
    us = up_index[0].astype(jnp.int32).reshape(NWH, NB, IB, CH)
    ud = up_index[1].astype(jnp.int32).reshape(NWH, NB, IB, CH)
    ds_ = down_index[0].astype(jnp.int32).reshape(NWH, NB, IB, CH)
    dd = down_index[1].astype(jnp.int32).reshape(NWH, NB, IB, CH)
    ua = up_attr.reshape(NWH, EPW, D)
    da = down_attr.reshape(NWH, EPW, D)
    zeros = jnp.zeros((RPT, D), jnp.float32)

    mesh = plsc.VectorSubcoreMesh(core_axis_name="c", subcore_axis_name="s")
    scatter = pl.kernel(
        _sc_body,
        mesh=mesh,
        out_type=[jax.ShapeDtypeStruct((N_PAD, D), jnp.float32),
                  jax.ShapeDtypeStruct((N_PAD, D), jnp.float32)],
        scratch_types=[
            pltpu.VMEM_SHARED((N_PAD, D), jnp.float32),
            pltpu.VMEM((IB, CH), jnp.int32),
            pltpu.VMEM((IB, CH), jnp.int32),
            pltpu.VMEM((IB, CH), jnp.int32),
            pltpu.VMEM((IB, CH), jnp.int32),
            pltpu.VMEM((CH, D), jnp.float32),
            pltpu.VMEM((CH, D), jnp.float32),
            pltpu.VMEM((CH, D), jnp.float32),
            pltpu.VMEM((CH, D), jnp.float32),
            pltpu.SemaphoreType.DMA,
            pltpu.SemaphoreType.DMA,
            pltpu.SemaphoreType.DMA,
            pltpu.SemaphoreType.DMA,
        ],
    )
    a0, a1 = scatter(x, us, ud, ua, ds_, dd, da, zeros)

    out = pl.pallas_call(
        _combine_body,
        grid=(1,),
        in_specs=[pl.BlockSpec((N, D), lambda i: (0, 0))] * 3,
        out_specs=pl.BlockSpec((N, D), lambda i: (0, 0)),
        out_shape=jax.ShapeDtypeStruct((N, D), jnp.float32),
    )(x, a0, a1)
    return out


# speedup vs baseline: 1.2189x; 1.0168x over previous
import jax
import jax.numpy as jnp
from jax import lax
from jax.experimental import pallas as pl
from jax.experimental.pallas import tpu as pltpu
from jax.experimental.pallas import tpu_sc as plsc

N = 10000
E = 320000
D = 128

NC = 2
NS = 16
NW = NC * NS
NWH = NW // 2
EPW = E // NWH
CH = 80
NCH = EPW // CH
IB = 10
NB = NCH // IB
N_PAD = 10112
RPT = N_PAD // NS


def _sc_body(x_hbm, us_hbm, ud_hbm, ua_hbm, ds_hbm, dd_hbm, da_hbm, z_hbm,
             out0, out1,
             acc, ia_src, ia_dst, ib_src, ib_dst, xb0, ab0, xb1, ab1,
             sem_g, sem_a, sem_s, sem_i):
    c = lax.axis_index("c")
    s = lax.axis_index("s")
    w = s * NC + c
    t = s

    pltpu.sync_copy(z_hbm, acc.at[pl.ds(t * RPT, RPT)])
    plsc.subcore_barrier()

    def run_pipeline(src_hbm, dst_hbm, attr_hbm, wl):
        def start_x(b, j, i_src, xb):
            pltpu.async_copy(x_hbm.at[i_src.at[j]], xb, sem_g)

        def start_a(b, j, ab):
            pltpu.async_copy(
                attr_hbm.at[wl, pl.ds((b * IB + j) * CH, CH)], ab, sem_a)

        def start(b, j, i_src, xb, ab):
            start_x(b, j, i_src, xb)
            start_a(b, j, ab)

        def wait_g_x(j, i_src, xb):
            pltpu.make_async_copy(x_hbm.at[i_src.at[j]], xb, sem_g).wait()

        def wait_g_a(b, j, ab):
            pltpu.make_async_copy(
                attr_hbm.at[wl, pl.ds((b * IB + j) * CH, CH)], ab,
                sem_a).wait()

        def scatter_x(j, i_dst, xb):
            pltpu.async_copy(xb, acc.at[i_dst.at[j]], sem_s, add=True)

        def scatter_a(j, i_dst, ab):
            pltpu.async_copy(ab, acc.at[i_dst.at[j]], sem_s, add=True)

        def wait_then_scatter(b, j, i_src, i_dst, xb, ab):
            wait_g_x(j, i_src, xb)
            scatter_x(j, i_dst, xb)
            wait_g_a(b, j, ab)
            scatter_a(j, i_dst, ab)

        def drain_x(j, i_dst, xb):
            pltpu.make_async_copy(xb, acc.at[i_dst.at[j]], sem_s).wait()

        def drain_a(j, i_dst, ab):
            pltpu.make_async_copy(ab, acc.at[i_dst.at[j]], sem_s).wait()

        def drain_scatters(j, i_dst, xb, ab):
            drain_x(j, i_dst, xb)
            drain_a(j, i_dst, ab)

        def prefetch_idx(b, i_src, i_dst):
            pltpu.async_copy(src_hbm.at[wl, b], i_src, sem_i)
            pltpu.async_copy(dst_hbm.at[wl, b], i_dst, sem_i)

        def wait_idx(b, i_src, i_dst):
            pltpu.make_async_copy(src_hbm.at[wl, b], i_src, sem_i).wait()
            pltpu.make_async_copy(dst_hbm.at[wl, b], i_dst, sem_i).wait()

        def process_block(b, i_src, i_dst, primed):
            def pair_body(i, cr):
                @pl.when(i > 0)
                def _():
                    drain_x(2 * i - 1, i_dst, xb1)
                start_x(b, 2 * i + 1, i_src, xb1)

                @pl.when(i > 0)
                def _():
                    drain_a(2 * i - 1, i_dst, ab1)
                start_a(b, 2 * i + 1, ab1)
                wait_then_scatter(b, 2 * i, i_src, i_dst, xb0, ab0)

                @pl.when(i < IB // 2 - 1)
                def _():
                    drain_x(2 * i, i_dst, xb0)
                    start_x(b, 2 * i + 2, i_src, xb0)
                    drain_a(2 * i, i_dst, ab0)
                    start_a(b, 2 * i + 2, ab0)
                wait_then_scatter(b, 2 * i + 1, i_src, i_dst, xb1, ab1)
                return cr

            if not primed:
                start(b, 0, i_src, xb0, ab0)
            lax.fori_loop(0, IB // 2, pair_body, 0)

        def finish_block(b, i_dst, nxt_b, nxt_src, prime):
            drain_x(IB - 2, i_dst, xb0)
            if prime:
                start_x(nxt_b, 0, nxt_src, xb0)
            drain_a(IB - 2, i_dst, ab0)
            if prime:
                start_a(nxt_b, 0, ab0)
            drain_scatters(IB - 1, i_dst, xb1, ab1)

        pltpu.sync_copy(src_hbm.at[wl, 0], ia_src)
        pltpu.sync_copy(dst_hbm.at[wl, 0], ia_dst)
        prefetch_idx(1, ib_src, ib_dst)
        process_block(0, ia_src, ia_dst, False)
        wait_idx(1, ib_src, ib_dst)
        finish_block(0, ia_dst, 1, ib_src, True)
        prefetch_idx(2, ia_src, ia_dst)

        def block_pair(p, carry):
            b1 = 2 * p + 1
            process_block(b1, ib_src, ib_dst, True)
            wait_idx(b1 + 1, ia_src, ia_dst)
            finish_block(b1, ib_dst, b1 + 1, ia_src, True)

            @pl.when(p < NB // 2 - 1)
            def _():
                prefetch_idx(b1 + 2, ib_src, ib_dst)
            b2 = b1 + 1
            process_block(b2, ia_src, ia_dst, True)

            @pl.when(p < NB // 2 - 1)
            def _():
                wait_idx(b2 + 1, ib_src, ib_dst)
                finish_block(b2, ia_dst, b2 + 1, ib_src, True)
                prefetch_idx(b2 + 2, ia_src, ia_dst)

            @pl.when(p == NB // 2 - 1)
            def _():
                finish_block(b2, ia_dst, 0, ia_src, False)
            return carry

        lax.fori_loop(0, NB // 2, block_pair, 0)

    @pl.when(w < NWH)
    def _():
        run_pipeline(us_hbm, ud_hbm, ua_hbm, w)

    @pl.when(w >= NWH)
    def _():
        run_pipeline(ds_hbm, dd_hbm, da_hbm, w - NWH)

    plsc.subcore_barrier()

    @pl.when(c == 0)
    def _():
        pltpu.sync_copy(acc.at[pl.ds(t * RPT, RPT)],
                        out0.at[pl.ds(t * RPT, RPT)])

    @pl.when(c == 1)
    def _():
        pltpu.sync_copy(acc.at[pl.ds(t * RPT, RPT)],
                        out1.at[pl.ds(t * RPT, RPT)])


def _combine_body(x_ref, a_ref, b_ref, o_ref):
    o_ref[...] = x_ref[...] + a_ref[...] + b_ref[...]


def kernel(x, up_index, up_attr, down_index, down_attr):
    us = up_index[0].astype(jnp.int32).reshape(NWH, NB, IB, CH)
    ud = up_index[1].astype(jnp.int32).reshape(NWH, NB, IB, CH)
    ds_ = down_index[0].astype(jnp.int32).reshape(NWH, NB, IB, CH)
    dd = down_index[1].astype(jnp.int32).reshape(NWH, NB, IB, CH)
    ua = up_attr.reshape(NWH, EPW, D)
    da = down_attr.reshape(NWH, EPW, D)
    zeros = jnp.zeros((RPT, D), jnp.float32)

    mesh = plsc.VectorSubcoreMesh(core_axis_name="c", subcore_axis_name="s")
    scatter = pl.kernel(
        _sc_body,
        mesh=mesh,
        out_type=[jax.ShapeDtypeStruct((N_PAD, D), jnp.float32),
                  jax.ShapeDtypeStruct((N_PAD, D), jnp.float32)],
        scratch_types=[
            pltpu.VMEM_SHARED((N_PAD, D), jnp.float32),
            pltpu.VMEM((IB, CH), jnp.int32),
            pltpu.VMEM((IB, CH), jnp.int32),
            pltpu.VMEM((IB, CH), jnp.int32),
            pltpu.VMEM((IB, CH), jnp.int32),
            pltpu.VMEM((CH, D), jnp.float32),
            pltpu.VMEM((CH, D), jnp.float32),
            pltpu.VMEM((CH, D), jnp.float32),
            pltpu.VMEM((CH, D), jnp.float32),
            pltpu.SemaphoreType.DMA,
            pltpu.SemaphoreType.DMA,
            pltpu.SemaphoreType.DMA,
            pltpu.SemaphoreType.DMA,
        ],
    )
    a0, a1 = scatter(x, us, ud, ua, ds_, dd, da, zeros)

    out = pl.pallas_call(
        _combine_body,
        grid=(1,),
        in_specs=[pl.BlockSpec((N, D), lambda i: (0, 0))] * 3,
        out_specs=pl.BlockSpec((N, D), lambda i: (0, 0)),
        out_shape=jax.ShapeDtypeStruct((N, D), jnp.float32),
    )(x, a0, a1)
    return out
